# Initial kernel scaffold; baseline (speedup 1.0000x reference)
import jax
import jax.numpy as jnp
from jax.experimental import pallas as pl


def kernel(x, dictionary):
    raise NotImplementedError("write your pallas kernel here")


# speedup vs baseline: 1.2694x; 1.2694x over previous
import functools

import jax
import jax.numpy as jnp
from jax import lax
from jax.experimental import pallas as pl
from jax.experimental.pallas import tpu as pltpu
from jax.experimental.pallas import tpu_sc as plsc


def _argmax_body(x_ref, tok_ref):
    xb = x_ref[...]
    m = jnp.max(xb, axis=-1, keepdims=True)
    ii = lax.broadcasted_iota(jnp.int32, xb.shape, 1)
    tok_ref[...] = jnp.min(jnp.where(xb == m, ii, xb.shape[-1]), axis=-1)


@functools.lru_cache(maxsize=None)
def _make_sc_gather(V, D, B):
    info = plsc.get_sparse_core_info()
    NC, NS = info.num_cores, info.num_subcores
    NW = NC * NS
    b_per_w = B // NW
    assert B % (8 * NW) == 0 and D % info.num_lanes == 0
    mesh = plsc.VectorSubcoreMesh(core_axis_name="c", subcore_axis_name="s")

    @functools.partial(
        pl.kernel,
        mesh=mesh,
        out_type=jax.ShapeDtypeStruct((B, D), jnp.float32),
        scratch_types=[
            pltpu.VMEM((b_per_w,), jnp.int32),
            pltpu.VMEM((b_per_w, D), jnp.float32),
            pltpu.SemaphoreType.DMA,
        ],
    )
    def gather(table_hbm, idx_hbm, out_hbm, idx_v, rows_v, sem):
        wid = lax.axis_index("s") * NC + lax.axis_index("c")
        base = wid * b_per_w
        pltpu.sync_copy(idx_hbm.at[pl.ds(base, b_per_w)], idx_v)
        pltpu.async_copy(table_hbm.at[idx_v], rows_v, sem).wait()
        pltpu.sync_copy(rows_v, out_hbm.at[pl.ds(base, b_per_w)])

    return gather


@jax.jit
def kernel(x, dictionary):
    B, N, V = x.shape
    D = dictionary.shape[1]
    R = B * N
    xf = x.reshape(R, V)

    BLK = 128
    tokens = pl.pallas_call(
        _argmax_body,
        grid=(R // BLK,),
        in_specs=[pl.BlockSpec((BLK, V), lambda i: (i, 0))],
        out_specs=pl.BlockSpec((BLK,), lambda i: (i,)),
        out_shape=jax.ShapeDtypeStruct((R,), jnp.int32),
    )(xf)

    DP = 128
    table = jnp.pad(dictionary, ((0, 0), (0, DP - D)))
    out = _make_sc_gather(V, DP, R)(table, tokens)
    return out[:, :D].reshape(B, N, D)


# speedup vs baseline: 1.5620x; 1.2305x over previous
import functools

import jax
import jax.numpy as jnp
from jax import lax
from jax.experimental import pallas as pl
from jax.experimental.pallas import tpu as pltpu
from jax.experimental.pallas import tpu_sc as plsc


def _argmax_body(x_ref, tok_ref):
    xb = x_ref[...]
    m = jnp.max(xb, axis=-1, keepdims=True)
    ii = lax.broadcasted_iota(jnp.int32, xb.shape, 1)
    tok_ref[...] = jnp.min(jnp.where(xb == m, ii, xb.shape[-1]), axis=-1)


@functools.lru_cache(maxsize=None)
def _make_sc_gather(V, D, B):
    info = plsc.get_sparse_core_info()
    NC, NS = info.num_cores, info.num_subcores
    NW = NC * NS
    b_per_w = B // NW
    assert B % (8 * NW) == 0 and D % info.num_lanes == 0
    mesh = plsc.VectorSubcoreMesh(core_axis_name="c", subcore_axis_name="s")

    @functools.partial(
        pl.kernel,
        mesh=mesh,
        out_type=jax.ShapeDtypeStruct((B, D), jnp.float32),
        scratch_types=[
            pltpu.VMEM((b_per_w,), jnp.int32),
            pltpu.VMEM((b_per_w, D), jnp.float32),
            pltpu.SemaphoreType.DMA,
        ],
    )
    def gather(table_hbm, idx_hbm, out_hbm, idx_v, rows_v, sem):
        wid = lax.axis_index("s") * NC + lax.axis_index("c")
        base = wid * b_per_w
        pltpu.sync_copy(idx_hbm.at[pl.ds(base, b_per_w)], idx_v)
        pltpu.async_copy(table_hbm.at[idx_v], rows_v, sem).wait()
        pltpu.sync_copy(rows_v, out_hbm.at[pl.ds(base, b_per_w)])

    return gather


@jax.jit
def kernel(x, dictionary):
    B, N, V = x.shape
    D = dictionary.shape[1]
    R = B * N
    xf = x.reshape(R, V)

    BLK = 512
    tokens = pl.pallas_call(
        _argmax_body,
        grid=(R // BLK,),
        in_specs=[pl.BlockSpec((BLK, V), lambda i: (i, 0))],
        out_specs=pl.BlockSpec((BLK,), lambda i: (i,)),
        out_shape=jax.ShapeDtypeStruct((R,), jnp.int32),
    )(xf)

    DP = 128
    table = jnp.pad(dictionary, ((0, 0), (0, DP - D)))
    out = _make_sc_gather(V, DP, R)(table, tokens)
    return out[:, :D].reshape(B, N, D)
